# baseline (speedup 1.0000x reference)
import functools

import jax
import jax.numpy as jnp
from jax import lax
from jax.experimental import pallas as pl
from jax.experimental.pallas import tpu as pltpu
from jax.experimental.pallas import tpu_sc as plsc

DIM = 16
L = 16
NC, NS = 2, 16
NW = NC * NS


def _rsqrt_newton(s):
    i = lax.bitcast_convert_type(s, jnp.int32)
    i = jnp.int32(0x5F3759DF) - lax.shift_right_arithmetic(i, 1)
    y = lax.bitcast_convert_type(i, jnp.float32)
    for _ in range(3):
        y = y * (jnp.float32(1.5) - jnp.float32(0.5) * s * y * y)
    return y


def _make_sc_loss(batch):
    bpw = batch // NW
    nchunk = bpw // L
    mesh = plsc.VectorSubcoreMesh(core_axis_name="c", subcore_axis_name="s")

    @functools.partial(
        pl.kernel,
        mesh=mesh,
        out_type=jax.ShapeDtypeStruct((NW, L), jnp.float32),
        scratch_types=[
            pltpu.VMEM((2 * bpw,), jnp.int32),
            pltpu.VMEM((2 * bpw, DIM), jnp.float32),
            pltpu.VMEM((bpw,), jnp.float32),
            pltpu.VMEM((L,), jnp.float32),
            pltpu.SemaphoreType.DMA,
        ],
        compiler_params=pltpu.CompilerParams(
            needs_layout_passes=False, use_tc_tiling_on_sc=False
        ),
    )
    def sc_loss(x_hbm, deg_hbm, emb_hbm, out_hbm, idx_v, rows_v, deg_v,
                acc_v, sem):
        wid = lax.axis_index("s") * NC + lax.axis_index("c")
        base = wid * bpw
        pltpu.sync_copy(x_hbm.at[pl.ds(2 * base, 2 * bpw)], idx_v)
        gather = pltpu.async_copy(emb_hbm.at[idx_v], rows_v, sem)
        pltpu.sync_copy(deg_hbm.at[pl.ds(base, bpw)], deg_v)
        gather.wait()

        lane = lax.iota(jnp.int32, L)

        def chunk_body(k, acc):
            even = 2 * (k * L + lane)
            odd = even + 1
            ssum = jnp.zeros((L,), jnp.float32)
            for d in range(DIM):
                col = jnp.full((L,), d, jnp.int32)
                a = plsc.load_gather(rows_v, [even, col])
                b = plsc.load_gather(rows_v, [odd, col])
                diff = a - b
                ssum = ssum + diff * diff
            ssum = jnp.maximum(ssum, jnp.float32(1e-30))
            dist = ssum * _rsqrt_newton(ssum)
            rate = jnp.exp(-dist)
            deg = deg_v[pl.ds(k * L, L)]
            return acc + deg * dist + rate

        acc = lax.fori_loop(0, nchunk, chunk_body,
                            jnp.zeros((L,), jnp.float32))
        acc_v[...] = acc
        pltpu.sync_copy(acc_v, out_hbm.at[wid])

    return sc_loss


@jax.jit
def kernel(x, degrees, embs):
    batch = x.shape[0]
    x_flat = x.astype(jnp.int32).reshape(-1)
    partials = _make_sc_loss(batch)(x_flat, degrees, embs)
    return jnp.sum(partials)
